# Initial kernel scaffold; baseline (speedup 1.0000x reference)
import jax
import jax.numpy as jnp
from jax.experimental import pallas as pl


def kernel(x, edge_index, W1, b1, W2, b2):
    raise NotImplementedError("write your pallas kernel here")


# speedup vs baseline: 28.0975x; 28.0975x over previous
import jax
import jax.numpy as jnp
from jax import lax
from jax.experimental import pallas as pl
from jax.experimental.pallas import tpu as pltpu
from jax.experimental.pallas import tpu_sc as plsc

N = 10000
N_PAD = 10240
DUMMY = N
D_FEAT = 128
H = 20
HP = 32
Z = 4
ZP = 16
DW = 16
E = 320000
NC, NS = 2, 16
NW = NC * NS
CH = 128
KF = 8
C = 80
E_PAD = NW * C * CH
T_ROWS = N_PAD // NS

_ROW_BLK = 1280
_GRID = N_PAD // _ROW_BLK


def _mesh():
    return plsc.VectorSubcoreMesh(
        core_axis_name="c", subcore_axis_name="s", num_cores=NC, num_subcores=NS
    )


_SC_PARAMS = pltpu.CompilerParams(use_tc_tiling_on_sc=False)


def _make_sc_degree():
    def body(dst3, ones_c, zrow, out, idx_d, ones_v, stage, acc, sem):
        cid = lax.axis_index("c")
        sid = lax.axis_index("s")
        t = cid * NS + sid
        pltpu.sync_copy(zrow, stage)
        pltpu.sync_copy(stage, acc.at[pl.ds(sid * T_ROWS, T_ROWS)])
        pltpu.sync_copy(ones_c, ones_v)
        pltpu.sync_copy(dst3.at[t], idx_d)
        plsc.subcore_barrier()

        @pl.loop(0, C // KF)
        def _grp(g):
            base = g * KF
            cps = [
                pltpu.async_copy(ones_v, acc.at[idx_d.at[base + b]], sem, add=True)
                for b in range(KF)
            ]
            for cp in cps:
                cp.wait()

        plsc.subcore_barrier()
        pltpu.sync_copy(acc.at[pl.ds(sid * T_ROWS, T_ROWS)], stage)
        pltpu.sync_copy(stage, out.at[cid, pl.ds(sid * T_ROWS, T_ROWS)])

    return pl.kernel(
        body,
        out_type=jax.ShapeDtypeStruct((NC, N_PAD, DW), jnp.float32),
        mesh=_mesh(),
        compiler_params=_SC_PARAMS,
        scratch_types=[
            pltpu.VMEM((C, CH), jnp.int32),
            pltpu.VMEM((CH, DW), jnp.float32),
            pltpu.VMEM((T_ROWS, DW), jnp.float32),
            pltpu.VMEM_SHARED((N_PAD, DW), jnp.float32),
            pltpu.SemaphoreType.DMA,
        ],
    )


def _make_sc_scatter(d):
    def body(src3, dst3, table, zrow, out, idx_s, idx_d, rows, stage, acc, sem):
        cid = lax.axis_index("c")
        sid = lax.axis_index("s")
        t = cid * NS + sid
        pltpu.sync_copy(zrow, stage)
        pltpu.sync_copy(stage, acc.at[pl.ds(sid * T_ROWS, T_ROWS)])
        pltpu.sync_copy(src3.at[t], idx_s)
        pltpu.sync_copy(dst3.at[t], idx_d)
        plsc.subcore_barrier()

        @pl.loop(0, C // KF)
        def _grp(g):
            base = g * KF
            cps = [
                pltpu.async_copy(table.at[idx_s.at[base + b]], rows.at[b], sem)
                for b in range(KF)
            ]
            for cp in cps:
                cp.wait()
            cps2 = [
                pltpu.async_copy(rows.at[b], acc.at[idx_d.at[base + b]], sem, add=True)
                for b in range(KF)
            ]
            for cp in cps2:
                cp.wait()

        plsc.subcore_barrier()
        pltpu.sync_copy(acc.at[pl.ds(sid * T_ROWS, T_ROWS)], stage)
        pltpu.sync_copy(stage, out.at[cid, pl.ds(sid * T_ROWS, T_ROWS)])

    return pl.kernel(
        body,
        out_type=jax.ShapeDtypeStruct((NC, N_PAD, d), jnp.float32),
        mesh=_mesh(),
        compiler_params=_SC_PARAMS,
        scratch_types=[
            pltpu.VMEM((C, CH), jnp.int32),
            pltpu.VMEM((C, CH), jnp.int32),
            pltpu.VMEM((KF, CH, d), jnp.float32),
            pltpu.VMEM((T_ROWS, d), jnp.float32),
            pltpu.VMEM_SHARED((N_PAD, d), jnp.float32),
            pltpu.SemaphoreType.DMA,
        ],
    )


_sc_degree = _make_sc_degree()
_sc_scatter_h = _make_sc_scatter(HP)
_sc_scatter_z = _make_sc_scatter(ZP)


def _tc_matmul(x_pad, W1):
    def body(x_ref, w_ref, o_ref):
        o_ref[...] = jnp.dot(x_ref[...], w_ref[...], preferred_element_type=jnp.float32)

    return pl.pallas_call(
        body,
        grid=(_GRID,),
        in_specs=[
            pl.BlockSpec((_ROW_BLK, D_FEAT), lambda i: (i, 0)),
            pl.BlockSpec((D_FEAT, HP), lambda i: (0, 0)),
        ],
        out_specs=pl.BlockSpec((_ROW_BLK, HP), lambda i: (i, 0)),
        out_shape=jax.ShapeDtypeStruct((N_PAD, HP), jnp.float32),
    )(x_pad, W1)


def _tc_scale(dega, degb, u1):
    def body(da, db, u, dinv_ref, v_ref):
        deg = 1.0 + da[...][:, :1] + db[...][:, :1]
        dinv = 1.0 / jnp.sqrt(deg)
        dinv_ref[...] = dinv
        v_ref[...] = u[...] * dinv

    return pl.pallas_call(
        body,
        grid=(_GRID,),
        in_specs=[
            pl.BlockSpec((_ROW_BLK, DW), lambda i: (i, 0)),
            pl.BlockSpec((_ROW_BLK, DW), lambda i: (i, 0)),
            pl.BlockSpec((_ROW_BLK, HP), lambda i: (i, 0)),
        ],
        out_specs=[
            pl.BlockSpec((_ROW_BLK, 1), lambda i: (i, 0)),
            pl.BlockSpec((_ROW_BLK, HP), lambda i: (i, 0)),
        ],
        out_shape=[
            jax.ShapeDtypeStruct((N_PAD, 1), jnp.float32),
            jax.ShapeDtypeStruct((N_PAD, HP), jnp.float32),
        ],
    )(dega, degb, u1)


def _tc_mid(dinv, sa, sb, v1, W2p, b1r):
    def body(di, a_ref, b_ref, v_ref, w_ref, bias_ref, o_ref):
        dinv_blk = di[...]
        h = dinv_blk * (a_ref[...] + b_ref[...] + v_ref[...]) + bias_ref[...]
        u2 = jnp.dot(h, w_ref[...], preferred_element_type=jnp.float32)
        o_ref[...] = u2 * dinv_blk

    return pl.pallas_call(
        body,
        grid=(_GRID,),
        in_specs=[
            pl.BlockSpec((_ROW_BLK, 1), lambda i: (i, 0)),
            pl.BlockSpec((_ROW_BLK, HP), lambda i: (i, 0)),
            pl.BlockSpec((_ROW_BLK, HP), lambda i: (i, 0)),
            pl.BlockSpec((_ROW_BLK, HP), lambda i: (i, 0)),
            pl.BlockSpec((HP, ZP), lambda i: (0, 0)),
            pl.BlockSpec((1, HP), lambda i: (0, 0)),
        ],
        out_specs=pl.BlockSpec((_ROW_BLK, ZP), lambda i: (i, 0)),
        out_shape=jax.ShapeDtypeStruct((N_PAD, ZP), jnp.float32),
    )(dinv, sa, sb, v1, W2p, b1r)


def _tc_final(dinv, sa, sb, v2, b2r):
    def body(di, a_ref, b_ref, v_ref, bias_ref, o_ref):
        o_ref[...] = di[...] * (a_ref[...] + b_ref[...] + v_ref[...]) + bias_ref[...]

    return pl.pallas_call(
        body,
        grid=(_GRID,),
        in_specs=[
            pl.BlockSpec((_ROW_BLK, 1), lambda i: (i, 0)),
            pl.BlockSpec((_ROW_BLK, ZP), lambda i: (i, 0)),
            pl.BlockSpec((_ROW_BLK, ZP), lambda i: (i, 0)),
            pl.BlockSpec((_ROW_BLK, ZP), lambda i: (i, 0)),
            pl.BlockSpec((1, ZP), lambda i: (0, 0)),
        ],
        out_specs=pl.BlockSpec((_ROW_BLK, ZP), lambda i: (i, 0)),
        out_shape=jax.ShapeDtypeStruct((N_PAD, ZP), jnp.float32),
    )(dinv, sa, sb, v2, b2r)


def kernel(x, edge_index, W1, b1, W2, b2):
    src = edge_index[0].astype(jnp.int32)
    dst = edge_index[1].astype(jnp.int32)
    epad = jnp.full((E_PAD - E,), DUMMY, jnp.int32)
    src3 = jnp.concatenate([src, epad]).reshape(NW, C, CH)
    dst3 = jnp.concatenate([dst, epad]).reshape(NW, C, CH)
    x_pad = jnp.pad(x, ((0, N_PAD - N), (0, 0)))
    W1p = jnp.pad(W1, ((0, 0), (0, HP - H)))
    W2p = jnp.pad(W2, ((0, HP - H), (0, ZP - Z)))
    b1r = jnp.pad(b1, (0, HP - H)).reshape(1, HP)
    b2r = jnp.pad(b2, (0, ZP - Z)).reshape(1, ZP)
    ones_c = jnp.ones((CH, DW), jnp.float32)
    zdw = jnp.zeros((T_ROWS, DW), jnp.float32)
    zh = jnp.zeros((T_ROWS, HP), jnp.float32)
    zz = jnp.zeros((T_ROWS, ZP), jnp.float32)

    deg2 = _sc_degree(dst3, ones_c, zdw)
    u1 = _tc_matmul(x_pad, W1p)
    dinv, v1 = _tc_scale(deg2[0], deg2[1], u1)
    s1 = _sc_scatter_h(src3, dst3, v1, zh)
    v2 = _tc_mid(dinv, s1[0], s1[1], v1, W2p, b1r)
    s2 = _sc_scatter_z(src3, dst3, v2, zz)
    zf = _tc_final(dinv, s2[0], s2[1], v2, b2r)
    return zf[:N, :Z]


# speedup vs baseline: 43.7643x; 1.5576x over previous
import jax
import jax.numpy as jnp
from jax import lax
from jax.experimental import pallas as pl
from jax.experimental.pallas import tpu as pltpu
from jax.experimental.pallas import tpu_sc as plsc

N = 10000
D_FEAT = 128
H = 20
HP = 32
Z = 4
ZP = 16
DW = 16
E = 320000
NC, NS = 2, 16
CH = 125
KF = 8
NCHUNK = E // CH
C0 = 112
C1 = 48
T_ROWS = N // NS

_ROW_BLK = 1000
_GRID = N // _ROW_BLK


def _mesh():
    return plsc.VectorSubcoreMesh(
        core_axis_name="c", subcore_axis_name="s", num_cores=NC, num_subcores=NS
    )


_SC_PARAMS = pltpu.CompilerParams(use_tc_tiling_on_sc=False)


def _stage_chunks(cid, sid, srcf, idx_ref):
    @pl.when(cid == 0)
    def _():
        pltpu.sync_copy(srcf.at[pl.ds(sid * C0, C0)], idx_ref.at[pl.ds(0, C0)])

    @pl.when(cid == 1)
    def _():
        pltpu.sync_copy(
            srcf.at[pl.ds(NS * C0 + sid * C1, C1)], idx_ref.at[pl.ds(0, C1)]
        )


def _make_sc_degree():
    def body(dstf, ones_c, zrow, out, idx_d, ones_v, stage, acc, sem):
        cid = lax.axis_index("c")
        sid = lax.axis_index("s")
        ngrp = jnp.where(cid == 0, C0 // KF, C1 // KF)
        pltpu.sync_copy(zrow, stage)
        pltpu.sync_copy(stage, acc.at[pl.ds(sid * T_ROWS, T_ROWS)])
        pltpu.sync_copy(ones_c, ones_v)
        _stage_chunks(cid, sid, dstf, idx_d)
        plsc.subcore_barrier()

        @pl.loop(0, ngrp)
        def _grp(g):
            base = g * KF
            cps = [
                pltpu.async_copy(ones_v, acc.at[idx_d.at[base + b]], sem, add=True)
                for b in range(KF)
            ]
            for cp in cps:
                cp.wait()

        plsc.subcore_barrier()
        pltpu.sync_copy(acc.at[pl.ds(sid * T_ROWS, T_ROWS)], stage)
        pltpu.sync_copy(stage, out.at[cid, pl.ds(sid * T_ROWS, T_ROWS)])

    return pl.kernel(
        body,
        out_type=jax.ShapeDtypeStruct((NC, N, DW), jnp.float32),
        mesh=_mesh(),
        compiler_params=_SC_PARAMS,
        scratch_types=[
            pltpu.VMEM((C0, CH), jnp.int32),
            pltpu.VMEM((CH, DW), jnp.float32),
            pltpu.VMEM((T_ROWS, DW), jnp.float32),
            pltpu.VMEM_SHARED((N, DW), jnp.float32),
            pltpu.SemaphoreType.DMA,
        ],
    )


def _make_sc_scatter(d):
    def body(srcf, dstf, table, zrow, out, idx_s, idx_d, rows, stage, acc, sem):
        cid = lax.axis_index("c")
        sid = lax.axis_index("s")
        ngrp = jnp.where(cid == 0, C0 // KF, C1 // KF)
        pltpu.sync_copy(zrow, stage)
        pltpu.sync_copy(stage, acc.at[pl.ds(sid * T_ROWS, T_ROWS)])
        _stage_chunks(cid, sid, srcf, idx_s)
        _stage_chunks(cid, sid, dstf, idx_d)
        plsc.subcore_barrier()

        @pl.loop(0, ngrp)
        def _grp(g):
            base = g * KF
            cps = [
                pltpu.async_copy(table.at[idx_s.at[base + b]], rows.at[b], sem)
                for b in range(KF)
            ]
            for cp in cps:
                cp.wait()
            cps2 = [
                pltpu.async_copy(rows.at[b], acc.at[idx_d.at[base + b]], sem, add=True)
                for b in range(KF)
            ]
            for cp in cps2:
                cp.wait()

        plsc.subcore_barrier()
        pltpu.sync_copy(acc.at[pl.ds(sid * T_ROWS, T_ROWS)], stage)
        pltpu.sync_copy(stage, out.at[cid, pl.ds(sid * T_ROWS, T_ROWS)])

    return pl.kernel(
        body,
        out_type=jax.ShapeDtypeStruct((NC, N, d), jnp.float32),
        mesh=_mesh(),
        compiler_params=_SC_PARAMS,
        scratch_types=[
            pltpu.VMEM((C0, CH), jnp.int32),
            pltpu.VMEM((C0, CH), jnp.int32),
            pltpu.VMEM((KF, CH, d), jnp.float32),
            pltpu.VMEM((T_ROWS, d), jnp.float32),
            pltpu.VMEM_SHARED((N, d), jnp.float32),
            pltpu.SemaphoreType.DMA,
        ],
    )


_sc_degree = _make_sc_degree()
_sc_scatter_h = _make_sc_scatter(HP)
_sc_scatter_z = _make_sc_scatter(ZP)


def _tc_matmul(x, W1p):
    def body(x_ref, w_ref, o_ref):
        o_ref[...] = jnp.dot(x_ref[...], w_ref[...], preferred_element_type=jnp.float32)

    return pl.pallas_call(
        body,
        grid=(_GRID,),
        in_specs=[
            pl.BlockSpec((_ROW_BLK, D_FEAT), lambda i: (i, 0)),
            pl.BlockSpec((D_FEAT, HP), lambda i: (0, 0)),
        ],
        out_specs=pl.BlockSpec((_ROW_BLK, HP), lambda i: (i, 0)),
        out_shape=jax.ShapeDtypeStruct((N, HP), jnp.float32),
    )(x, W1p)


def _tc_scale(deg2, u1):
    def body(dref, u, dinv_ref, v_ref):
        dd = dref[...]
        deg = 1.0 + dd[0][:, :1] + dd[1][:, :1]
        dinv = 1.0 / jnp.sqrt(deg)
        dinv_ref[...] = dinv
        v_ref[...] = u[...] * dinv

    return pl.pallas_call(
        body,
        grid=(_GRID,),
        in_specs=[
            pl.BlockSpec((NC, _ROW_BLK, DW), lambda i: (0, i, 0)),
            pl.BlockSpec((_ROW_BLK, HP), lambda i: (i, 0)),
        ],
        out_specs=[
            pl.BlockSpec((_ROW_BLK, 1), lambda i: (i, 0)),
            pl.BlockSpec((_ROW_BLK, HP), lambda i: (i, 0)),
        ],
        out_shape=[
            jax.ShapeDtypeStruct((N, 1), jnp.float32),
            jax.ShapeDtypeStruct((N, HP), jnp.float32),
        ],
    )(deg2, u1)


def _tc_mid(dinv, s1, v1, W2p, b1r):
    def body(di, s_ref, v_ref, w_ref, bias_ref, o_ref):
        dinv_blk = di[...]
        ss = s_ref[...]
        h = dinv_blk * (ss[0] + ss[1] + v_ref[...]) + bias_ref[...]
        u2 = jnp.dot(h, w_ref[...], preferred_element_type=jnp.float32)
        o_ref[...] = u2 * dinv_blk

    return pl.pallas_call(
        body,
        grid=(_GRID,),
        in_specs=[
            pl.BlockSpec((_ROW_BLK, 1), lambda i: (i, 0)),
            pl.BlockSpec((NC, _ROW_BLK, HP), lambda i: (0, i, 0)),
            pl.BlockSpec((_ROW_BLK, HP), lambda i: (i, 0)),
            pl.BlockSpec((HP, ZP), lambda i: (0, 0)),
            pl.BlockSpec((1, HP), lambda i: (0, 0)),
        ],
        out_specs=pl.BlockSpec((_ROW_BLK, ZP), lambda i: (i, 0)),
        out_shape=jax.ShapeDtypeStruct((N, ZP), jnp.float32),
    )(dinv, s1, v1, W2p, b1r)


def _tc_final(dinv, s2, v2, b2r):
    def body(di, s_ref, v_ref, bias_ref, o_ref):
        ss = s_ref[...]
        zfull = di[...] * (ss[0] + ss[1] + v_ref[...]) + bias_ref[...]
        o_ref[...] = zfull[:, :Z]

    return pl.pallas_call(
        body,
        grid=(_GRID,),
        in_specs=[
            pl.BlockSpec((_ROW_BLK, 1), lambda i: (i, 0)),
            pl.BlockSpec((NC, _ROW_BLK, ZP), lambda i: (0, i, 0)),
            pl.BlockSpec((_ROW_BLK, ZP), lambda i: (i, 0)),
            pl.BlockSpec((1, ZP), lambda i: (0, 0)),
        ],
        out_specs=pl.BlockSpec((_ROW_BLK, Z), lambda i: (i, 0)),
        out_shape=jax.ShapeDtypeStruct((N, Z), jnp.float32),
    )(dinv, s2, v2, b2r)


def kernel(x, edge_index, W1, b1, W2, b2):
    ei = edge_index.astype(jnp.int32)
    srcf = ei[0].reshape(NCHUNK, CH)
    dstf = ei[1].reshape(NCHUNK, CH)
    W1p = jnp.pad(W1, ((0, 0), (0, HP - H)))
    W2p = jnp.pad(W2, ((0, HP - H), (0, ZP - Z)))
    b1r = jnp.pad(b1, (0, HP - H)).reshape(1, HP)
    b2r = jnp.pad(b2, (0, ZP - Z)).reshape(1, ZP)
    ones_c = jnp.ones((CH, DW), jnp.float32)
    zdw = jnp.zeros((T_ROWS, DW), jnp.float32)
    zh = jnp.zeros((T_ROWS, HP), jnp.float32)
    zz = jnp.zeros((T_ROWS, ZP), jnp.float32)

    deg2 = _sc_degree(dstf, ones_c, zdw)
    u1 = _tc_matmul(x, W1p)
    dinv, v1 = _tc_scale(deg2, u1)
    s1 = _sc_scatter_h(srcf, dstf, v1, zh)
    v2 = _tc_mid(dinv, s1, v1, W2p, b1r)
    s2 = _sc_scatter_z(srcf, dstf, v2, zz)
    return _tc_final(dinv, s2, v2, b2r)


# speedup vs baseline: 49.5309x; 1.1318x over previous
import jax
import jax.numpy as jnp
from jax import lax
from jax.experimental import pallas as pl
from jax.experimental.pallas import tpu as pltpu
from jax.experimental.pallas import tpu_sc as plsc

N = 10000
D_FEAT = 128
H = 20
HP = 32
Z = 4
ZP = 16
DW = 16
E = 320000
NC, NS = 2, 16
CH = 125
KF = 8
NCHUNK = E // CH
C0 = 88
C1 = 72
T_ROWS = N // NS

_ROW_BLK = 1000
_GRID = N // _ROW_BLK


def _mesh():
    return plsc.VectorSubcoreMesh(
        core_axis_name="c", subcore_axis_name="s", num_cores=NC, num_subcores=NS
    )


_SC_PARAMS = pltpu.CompilerParams(use_tc_tiling_on_sc=False)


def _stage_chunks(cid, sid, e3, row, idx_ref):
    @pl.when(cid == 0)
    def _():
        pltpu.sync_copy(e3.at[row, pl.ds(sid * C0, C0)], idx_ref.at[pl.ds(0, C0)])

    @pl.when(cid == 1)
    def _():
        pltpu.sync_copy(
            e3.at[row, pl.ds(NS * C0 + sid * C1, C1)], idx_ref.at[pl.ds(0, C1)]
        )


def _make_sc_degree():
    def body(e3, ones_c, zrow, out, idx_d, ones_v, stage, acc, sem):
        cid = lax.axis_index("c")
        sid = lax.axis_index("s")
        ngrp = jnp.where(cid == 0, C0 // KF, C1 // KF)
        pltpu.sync_copy(zrow, stage)
        pltpu.sync_copy(stage, acc.at[pl.ds(sid * T_ROWS, T_ROWS)])
        pltpu.sync_copy(ones_c, ones_v)
        _stage_chunks(cid, sid, e3, 1, idx_d)
        plsc.subcore_barrier()

        @pl.loop(0, ngrp)
        def _grp(g):
            base = g * KF
            cps = [
                pltpu.async_copy(ones_v, acc.at[idx_d.at[base + b]], sem, add=True)
                for b in range(KF)
            ]
            for cp in cps:
                cp.wait()

        plsc.subcore_barrier()
        pltpu.sync_copy(acc.at[pl.ds(sid * T_ROWS, T_ROWS)], stage)
        pltpu.sync_copy(stage, out.at[cid, pl.ds(sid * T_ROWS, T_ROWS)])

    return pl.kernel(
        body,
        out_type=jax.ShapeDtypeStruct((NC, N, DW), jnp.float32),
        mesh=_mesh(),
        compiler_params=_SC_PARAMS,
        scratch_types=[
            pltpu.VMEM((C0, CH), jnp.int32),
            pltpu.VMEM((CH, DW), jnp.float32),
            pltpu.VMEM((T_ROWS, DW), jnp.float32),
            pltpu.VMEM_SHARED((N, DW), jnp.float32),
            pltpu.SemaphoreType.DMA,
        ],
    )


def _make_sc_scatter(d):
    def body(e3, table, zrow, out, idx_s, idx_d, rows, stage, acc, sem):
        cid = lax.axis_index("c")
        sid = lax.axis_index("s")
        ngrp = jnp.where(cid == 0, C0 // KF, C1 // KF)
        pltpu.sync_copy(zrow, stage)
        pltpu.sync_copy(stage, acc.at[pl.ds(sid * T_ROWS, T_ROWS)])
        _stage_chunks(cid, sid, e3, 0, idx_s)
        _stage_chunks(cid, sid, e3, 1, idx_d)
        plsc.subcore_barrier()

        @pl.loop(0, ngrp)
        def _grp(g):
            base = g * KF
            cps = [
                pltpu.async_copy(table.at[idx_s.at[base + b]], rows.at[b], sem)
                for b in range(KF)
            ]
            for cp in cps:
                cp.wait()
            cps2 = [
                pltpu.async_copy(rows.at[b], acc.at[idx_d.at[base + b]], sem, add=True)
                for b in range(KF)
            ]
            for cp in cps2:
                cp.wait()

        plsc.subcore_barrier()
        pltpu.sync_copy(acc.at[pl.ds(sid * T_ROWS, T_ROWS)], stage)
        pltpu.sync_copy(stage, out.at[cid, pl.ds(sid * T_ROWS, T_ROWS)])

    return pl.kernel(
        body,
        out_type=jax.ShapeDtypeStruct((NC, N, d), jnp.float32),
        mesh=_mesh(),
        compiler_params=_SC_PARAMS,
        scratch_types=[
            pltpu.VMEM((C0, CH), jnp.int32),
            pltpu.VMEM((C0, CH), jnp.int32),
            pltpu.VMEM((KF, CH, d), jnp.float32),
            pltpu.VMEM((T_ROWS, d), jnp.float32),
            pltpu.VMEM_SHARED((N, d), jnp.float32),
            pltpu.SemaphoreType.DMA,
        ],
    )


_sc_degree = _make_sc_degree()
_sc_scatter_h = _make_sc_scatter(HP)
_sc_scatter_z = _make_sc_scatter(ZP)


def _tc_matmul(x, W1p):
    def body(x_ref, w_ref, o_ref):
        o_ref[...] = jnp.dot(x_ref[...], w_ref[...], preferred_element_type=jnp.float32)

    return pl.pallas_call(
        body,
        grid=(_GRID,),
        in_specs=[
            pl.BlockSpec((_ROW_BLK, D_FEAT), lambda i: (i, 0)),
            pl.BlockSpec((D_FEAT, HP), lambda i: (0, 0)),
        ],
        out_specs=pl.BlockSpec((_ROW_BLK, HP), lambda i: (i, 0)),
        out_shape=jax.ShapeDtypeStruct((N, HP), jnp.float32),
    )(x, W1p)


def _tc_scale(deg2, u1):
    def body(dref, u, dinv_ref, v_ref):
        dd = dref[...]
        deg = 1.0 + dd[0][:, :1] + dd[1][:, :1]
        dinv = 1.0 / jnp.sqrt(deg)
        dinv_ref[...] = dinv
        v_ref[...] = u[...] * dinv

    return pl.pallas_call(
        body,
        grid=(_GRID,),
        in_specs=[
            pl.BlockSpec((NC, _ROW_BLK, DW), lambda i: (0, i, 0)),
            pl.BlockSpec((_ROW_BLK, HP), lambda i: (i, 0)),
        ],
        out_specs=[
            pl.BlockSpec((_ROW_BLK, 1), lambda i: (i, 0)),
            pl.BlockSpec((_ROW_BLK, HP), lambda i: (i, 0)),
        ],
        out_shape=[
            jax.ShapeDtypeStruct((N, 1), jnp.float32),
            jax.ShapeDtypeStruct((N, HP), jnp.float32),
        ],
    )(deg2, u1)


def _tc_mid(dinv, s1, v1, W2p, b1r):
    def body(di, s_ref, v_ref, w_ref, bias_ref, o_ref):
        dinv_blk = di[...]
        ss = s_ref[...]
        h = dinv_blk * (ss[0] + ss[1] + v_ref[...]) + bias_ref[...]
        u2 = jnp.dot(h, w_ref[...], preferred_element_type=jnp.float32)
        o_ref[...] = u2 * dinv_blk

    return pl.pallas_call(
        body,
        grid=(_GRID,),
        in_specs=[
            pl.BlockSpec((_ROW_BLK, 1), lambda i: (i, 0)),
            pl.BlockSpec((NC, _ROW_BLK, HP), lambda i: (0, i, 0)),
            pl.BlockSpec((_ROW_BLK, HP), lambda i: (i, 0)),
            pl.BlockSpec((HP, ZP), lambda i: (0, 0)),
            pl.BlockSpec((1, HP), lambda i: (0, 0)),
        ],
        out_specs=pl.BlockSpec((_ROW_BLK, ZP), lambda i: (i, 0)),
        out_shape=jax.ShapeDtypeStruct((N, ZP), jnp.float32),
    )(dinv, s1, v1, W2p, b1r)


def _tc_final(dinv, s2, v2, b2r):
    def body(di, s_ref, v_ref, bias_ref, o_ref):
        ss = s_ref[...]
        zfull = di[...] * (ss[0] + ss[1] + v_ref[...]) + bias_ref[...]
        o_ref[...] = zfull[:, :Z]

    return pl.pallas_call(
        body,
        grid=(_GRID,),
        in_specs=[
            pl.BlockSpec((_ROW_BLK, 1), lambda i: (i, 0)),
            pl.BlockSpec((NC, _ROW_BLK, ZP), lambda i: (0, i, 0)),
            pl.BlockSpec((_ROW_BLK, ZP), lambda i: (i, 0)),
            pl.BlockSpec((1, ZP), lambda i: (0, 0)),
        ],
        out_specs=pl.BlockSpec((_ROW_BLK, Z), lambda i: (i, 0)),
        out_shape=jax.ShapeDtypeStruct((N, Z), jnp.float32),
    )(dinv, s2, v2, b2r)


def kernel(x, edge_index, W1, b1, W2, b2):
    e3 = edge_index.astype(jnp.int32).reshape(2, NCHUNK, CH)
    W1p = jnp.pad(W1, ((0, 0), (0, HP - H)))
    W2p = jnp.pad(W2, ((0, HP - H), (0, ZP - Z)))
    b1r = jnp.pad(b1, (0, HP - H)).reshape(1, HP)
    b2r = jnp.pad(b2, (0, ZP - Z)).reshape(1, ZP)
    ones_c = jnp.ones((CH, DW), jnp.float32)
    zdw = jnp.zeros((T_ROWS, DW), jnp.float32)
    zh = jnp.zeros((T_ROWS, HP), jnp.float32)
    zz = jnp.zeros((T_ROWS, ZP), jnp.float32)

    deg2 = _sc_degree(e3, ones_c, zdw)
    u1 = _tc_matmul(x, W1p)
    dinv, v1 = _tc_scale(deg2, u1)
    s1 = _sc_scatter_h(e3, v1, zh)
    v2 = _tc_mid(dinv, s1, v1, W2p, b1r)
    s2 = _sc_scatter_z(e3, v2, zz)
    return _tc_final(dinv, s2, v2, b2r)


# speedup vs baseline: 57.3785x; 1.1584x over previous
import jax
import jax.numpy as jnp
from jax import lax
from jax.experimental import pallas as pl
from jax.experimental.pallas import tpu as pltpu
from jax.experimental.pallas import tpu_sc as plsc

N = 10000
D_FEAT = 128
H = 20
HP = 32
Z = 4
ZP = 16
DW = 16
E = 320000
NC, NS = 2, 16
CH = 125
KF = 8
NCHUNK = E // CH
C0 = 88
C1 = 72
T_ROWS = N // NS

_ROW_BLK = N
_GRID = N // _ROW_BLK


def _mesh():
    return plsc.VectorSubcoreMesh(
        core_axis_name="c", subcore_axis_name="s", num_cores=NC, num_subcores=NS
    )


_SC_PARAMS = pltpu.CompilerParams(use_tc_tiling_on_sc=False)


def _stage_chunks(cid, sid, e3, row, idx_ref):
    @pl.when(cid == 0)
    def _():
        pltpu.sync_copy(e3.at[row, pl.ds(sid * C0, C0)], idx_ref.at[pl.ds(0, C0)])

    @pl.when(cid == 1)
    def _():
        pltpu.sync_copy(
            e3.at[row, pl.ds(NS * C0 + sid * C1, C1)], idx_ref.at[pl.ds(0, C1)]
        )


def _make_sc_degree():
    def body(e3, ones_c, zrow, out, idx_d, ones_v, stage, acc, sem):
        cid = lax.axis_index("c")
        sid = lax.axis_index("s")
        ngrp = jnp.where(cid == 0, C0 // KF, C1 // KF)
        pltpu.sync_copy(zrow, stage)
        pltpu.sync_copy(stage, acc.at[pl.ds(sid * T_ROWS, T_ROWS)])
        pltpu.sync_copy(ones_c, ones_v)
        _stage_chunks(cid, sid, e3, 1, idx_d)
        plsc.subcore_barrier()

        @pl.loop(0, ngrp)
        def _grp(g):
            base = g * KF
            cps = [
                pltpu.async_copy(ones_v, acc.at[idx_d.at[base + b]], sem, add=True)
                for b in range(KF)
            ]
            for cp in cps:
                cp.wait()

        plsc.subcore_barrier()
        pltpu.sync_copy(acc.at[pl.ds(sid * T_ROWS, T_ROWS)], stage)
        pltpu.sync_copy(stage, out.at[cid, pl.ds(sid * T_ROWS, T_ROWS)])

    return pl.kernel(
        body,
        out_type=jax.ShapeDtypeStruct((NC, N, DW), jnp.float32),
        mesh=_mesh(),
        compiler_params=_SC_PARAMS,
        scratch_types=[
            pltpu.VMEM((C0, CH), jnp.int32),
            pltpu.VMEM((CH, DW), jnp.float32),
            pltpu.VMEM((T_ROWS, DW), jnp.float32),
            pltpu.VMEM_SHARED((N, DW), jnp.float32),
            pltpu.SemaphoreType.DMA,
        ],
    )


def _make_sc_scatter(d):
    def body(e3, table, zrow, out, idx_s, idx_d, rows, stage, acc, semA, semB, semS):
        cid = lax.axis_index("c")
        sid = lax.axis_index("s")
        ngrp = jnp.where(cid == 0, C0 // KF, C1 // KF)
        pltpu.sync_copy(zrow, stage)
        pltpu.sync_copy(stage, acc.at[pl.ds(sid * T_ROWS, T_ROWS)])
        _stage_chunks(cid, sid, e3, 0, idx_s)
        _stage_chunks(cid, sid, e3, 1, idx_d)
        plsc.subcore_barrier()

        def fire_gathers(g, buf, sem):
            for b in range(KF):
                pltpu.async_copy(table.at[idx_s.at[g * KF + b]], buf.at[b], sem)

        def drain_gathers(g, buf, sem):
            for b in range(KF):
                pltpu.make_async_copy(
                    table.at[idx_s.at[g * KF + b]], buf.at[b], sem
                ).wait()

        def run_scatters(g, buf):
            cps = [
                pltpu.async_copy(
                    buf.at[b], acc.at[idx_d.at[g * KF + b]], semS, add=True
                )
                for b in range(KF)
            ]
            for cp in cps:
                cp.wait()

        def step(g, buf, sem, obuf, osem):
            @pl.when(g + 1 < ngrp)
            def _():
                fire_gathers(g + 1, obuf, osem)

            drain_gathers(g, buf, sem)
            run_scatters(g, buf)

        fire_gathers(0, rows.at[0], semA)

        @pl.loop(0, ngrp)
        def _grp(g):
            @pl.when(g % 2 == 0)
            def _():
                step(g, rows.at[0], semA, rows.at[1], semB)

            @pl.when(g % 2 == 1)
            def _():
                step(g, rows.at[1], semB, rows.at[0], semA)

        plsc.subcore_barrier()
        pltpu.sync_copy(acc.at[pl.ds(sid * T_ROWS, T_ROWS)], stage)
        pltpu.sync_copy(stage, out.at[cid, pl.ds(sid * T_ROWS, T_ROWS)])

    return pl.kernel(
        body,
        out_type=jax.ShapeDtypeStruct((NC, N, d), jnp.float32),
        mesh=_mesh(),
        compiler_params=_SC_PARAMS,
        scratch_types=[
            pltpu.VMEM((C0, CH), jnp.int32),
            pltpu.VMEM((C0, CH), jnp.int32),
            pltpu.VMEM((2, KF, CH, d), jnp.float32),
            pltpu.VMEM((T_ROWS, d), jnp.float32),
            pltpu.VMEM_SHARED((N, d), jnp.float32),
            pltpu.SemaphoreType.DMA,
            pltpu.SemaphoreType.DMA,
            pltpu.SemaphoreType.DMA,
        ],
    )


_sc_degree = _make_sc_degree()
_sc_scatter_h = _make_sc_scatter(HP)
_sc_scatter_z = _make_sc_scatter(ZP)


def _tc_matmul(x, W1p):
    def body(x_ref, w_ref, o_ref):
        o_ref[...] = jnp.dot(x_ref[...], w_ref[...], preferred_element_type=jnp.float32)

    return pl.pallas_call(
        body,
        grid=(_GRID,),
        in_specs=[
            pl.BlockSpec((_ROW_BLK, D_FEAT), lambda i: (i, 0)),
            pl.BlockSpec((D_FEAT, HP), lambda i: (0, 0)),
        ],
        out_specs=pl.BlockSpec((_ROW_BLK, HP), lambda i: (i, 0)),
        out_shape=jax.ShapeDtypeStruct((N, HP), jnp.float32),
    )(x, W1p)


def _tc_scale(deg2, u1):
    def body(dref, u, dinv_ref, v_ref):
        dd = dref[...]
        deg = 1.0 + dd[0][:, :1] + dd[1][:, :1]
        dinv = 1.0 / jnp.sqrt(deg)
        dinv_ref[...] = dinv
        v_ref[...] = u[...] * dinv

    return pl.pallas_call(
        body,
        grid=(_GRID,),
        in_specs=[
            pl.BlockSpec((NC, _ROW_BLK, DW), lambda i: (0, i, 0)),
            pl.BlockSpec((_ROW_BLK, HP), lambda i: (i, 0)),
        ],
        out_specs=[
            pl.BlockSpec((_ROW_BLK, 1), lambda i: (i, 0)),
            pl.BlockSpec((_ROW_BLK, HP), lambda i: (i, 0)),
        ],
        out_shape=[
            jax.ShapeDtypeStruct((N, 1), jnp.float32),
            jax.ShapeDtypeStruct((N, HP), jnp.float32),
        ],
    )(deg2, u1)


def _tc_mid(dinv, s1, v1, W2p, b1r):
    def body(di, s_ref, v_ref, w_ref, bias_ref, o_ref):
        dinv_blk = di[...]
        ss = s_ref[...]
        h = dinv_blk * (ss[0] + ss[1] + v_ref[...]) + bias_ref[...]
        u2 = jnp.dot(h, w_ref[...], preferred_element_type=jnp.float32)
        o_ref[...] = u2 * dinv_blk

    return pl.pallas_call(
        body,
        grid=(_GRID,),
        in_specs=[
            pl.BlockSpec((_ROW_BLK, 1), lambda i: (i, 0)),
            pl.BlockSpec((NC, _ROW_BLK, HP), lambda i: (0, i, 0)),
            pl.BlockSpec((_ROW_BLK, HP), lambda i: (i, 0)),
            pl.BlockSpec((HP, ZP), lambda i: (0, 0)),
            pl.BlockSpec((1, HP), lambda i: (0, 0)),
        ],
        out_specs=pl.BlockSpec((_ROW_BLK, ZP), lambda i: (i, 0)),
        out_shape=jax.ShapeDtypeStruct((N, ZP), jnp.float32),
    )(dinv, s1, v1, W2p, b1r)


def _tc_final(dinv, s2, v2, b2r):
    def body(di, s_ref, v_ref, bias_ref, o_ref):
        ss = s_ref[...]
        zfull = di[...] * (ss[0] + ss[1] + v_ref[...]) + bias_ref[...]
        o_ref[...] = zfull[:, :Z]

    return pl.pallas_call(
        body,
        grid=(_GRID,),
        in_specs=[
            pl.BlockSpec((_ROW_BLK, 1), lambda i: (i, 0)),
            pl.BlockSpec((NC, _ROW_BLK, ZP), lambda i: (0, i, 0)),
            pl.BlockSpec((_ROW_BLK, ZP), lambda i: (i, 0)),
            pl.BlockSpec((1, ZP), lambda i: (0, 0)),
        ],
        out_specs=pl.BlockSpec((_ROW_BLK, Z), lambda i: (i, 0)),
        out_shape=jax.ShapeDtypeStruct((N, Z), jnp.float32),
    )(dinv, s2, v2, b2r)


def kernel(x, edge_index, W1, b1, W2, b2):
    e3 = edge_index.astype(jnp.int32).reshape(2, NCHUNK, CH)
    W1p = jnp.pad(W1, ((0, 0), (0, HP - H)))
    W2p = jnp.pad(W2, ((0, HP - H), (0, ZP - Z)))
    b1r = jnp.pad(b1, (0, HP - H)).reshape(1, HP)
    b2r = jnp.pad(b2, (0, ZP - Z)).reshape(1, ZP)
    ones_c = jnp.ones((CH, DW), jnp.float32)
    zdw = jnp.zeros((T_ROWS, DW), jnp.float32)
    zh = jnp.zeros((T_ROWS, HP), jnp.float32)
    zz = jnp.zeros((T_ROWS, ZP), jnp.float32)

    deg2 = _sc_degree(e3, ones_c, zdw)
    u1 = _tc_matmul(x, W1p)
    dinv, v1 = _tc_scale(deg2, u1)
    s1 = _sc_scatter_h(e3, v1, zh)
    v2 = _tc_mid(dinv, s1, v1, W2p, b1r)
    s2 = _sc_scatter_z(e3, v2, zz)
    return _tc_final(dinv, s2, v2, b2r)


# speedup vs baseline: 58.7795x; 1.0244x over previous
import jax
import jax.numpy as jnp
from jax import lax
from jax.experimental import pallas as pl
from jax.experimental.pallas import tpu as pltpu
from jax.experimental.pallas import tpu_sc as plsc

N = 10000
D_FEAT = 128
H = 20
HP = 32
Z = 4
ZP = 16
DW = 16
E = 320000
NC, NS = 2, 16
CH = 125
KF = 8
NCHUNK = E // CH
C0 = 88
C1 = 72
T_ROWS = N // NS

_ROW_BLK = N
_GRID = N // _ROW_BLK


def _mesh():
    return plsc.VectorSubcoreMesh(
        core_axis_name="c", subcore_axis_name="s", num_cores=NC, num_subcores=NS
    )


_SC_PARAMS = pltpu.CompilerParams(use_tc_tiling_on_sc=False)


def _stage_chunks(cid, sid, e3, row, idx_ref):
    @pl.when(cid == 0)
    def _():
        pltpu.sync_copy(e3.at[row, pl.ds(sid * C0, C0)], idx_ref.at[pl.ds(0, C0)])

    @pl.when(cid == 1)
    def _():
        pltpu.sync_copy(
            e3.at[row, pl.ds(NS * C0 + sid * C1, C1)], idx_ref.at[pl.ds(0, C1)]
        )


_DCH = 128
_DNCH = E // _DCH


def _make_sc_degree():
    def body(e2, ones_c, zrow, out, idx_d, ones_v, stage, acc, sem):
        cid = lax.axis_index("c")
        sid = lax.axis_index("s")
        t = cid * NS + sid
        cnt = 78 + (t < 4).astype(jnp.int32)
        base = (78 * t + jnp.minimum(t, 4)) * _DCH
        pltpu.sync_copy(zrow, stage)
        pltpu.sync_copy(stage, acc.at[pl.ds(sid * T_ROWS, T_ROWS)])
        pltpu.sync_copy(ones_c, ones_v)

        @pl.when(t < 4)
        def _():
            pltpu.sync_copy(
                e2.at[1, pl.ds(base, 79 * _DCH)], idx_d.at[pl.ds(0, 79 * _DCH)]
            )

        @pl.when(t >= 4)
        def _():
            pltpu.sync_copy(
                e2.at[1, pl.ds(base, 78 * _DCH)], idx_d.at[pl.ds(0, 78 * _DCH)]
            )

        plsc.subcore_barrier()

        @pl.loop(0, 10)
        def _grp(g):
            for b in range(KF):
                c = g * KF + b

                @pl.when(c < cnt)
                def _():
                    pltpu.async_copy(
                        ones_v, acc.at[idx_d.at[pl.ds(c * _DCH, _DCH)]], sem, add=True
                    )

            for b in range(KF):
                c = g * KF + b

                @pl.when(c < cnt)
                def _():
                    pltpu.make_async_copy(
                        ones_v, acc.at[idx_d.at[pl.ds(c * _DCH, _DCH)]], sem
                    ).wait()

        plsc.subcore_barrier()
        pltpu.sync_copy(acc.at[pl.ds(sid * T_ROWS, T_ROWS)], stage)
        pltpu.sync_copy(stage, out.at[cid, pl.ds(sid * T_ROWS, T_ROWS)])

    return pl.kernel(
        body,
        out_type=jax.ShapeDtypeStruct((NC, N, DW), jnp.float32),
        mesh=_mesh(),
        compiler_params=_SC_PARAMS,
        scratch_types=[
            pltpu.VMEM((79 * _DCH,), jnp.int32),
            pltpu.VMEM((_DCH, DW), jnp.float32),
            pltpu.VMEM((T_ROWS, DW), jnp.float32),
            pltpu.VMEM_SHARED((N, DW), jnp.float32),
            pltpu.SemaphoreType.DMA,
        ],
    )


def _make_sc_scatter(d):
    def body(e3, table, zrow, out, idx_s, idx_d, rows, stage, acc, semA, semB, semS):
        cid = lax.axis_index("c")
        sid = lax.axis_index("s")
        ngrp = jnp.where(cid == 0, C0 // KF, C1 // KF)
        pltpu.sync_copy(zrow, stage)
        pltpu.sync_copy(stage, acc.at[pl.ds(sid * T_ROWS, T_ROWS)])
        _stage_chunks(cid, sid, e3, 0, idx_s)
        _stage_chunks(cid, sid, e3, 1, idx_d)
        plsc.subcore_barrier()

        def fire_gathers(g, buf, sem):
            for b in range(KF):
                pltpu.async_copy(table.at[idx_s.at[g * KF + b]], buf.at[b], sem)

        def drain_gathers(g, buf, sem):
            for b in range(KF):
                pltpu.make_async_copy(
                    table.at[idx_s.at[g * KF + b]], buf.at[b], sem
                ).wait()

        def run_scatters(g, buf):
            cps = [
                pltpu.async_copy(
                    buf.at[b], acc.at[idx_d.at[g * KF + b]], semS, add=True
                )
                for b in range(KF)
            ]
            for cp in cps:
                cp.wait()

        def step(g, buf, sem, obuf, osem):
            @pl.when(g + 1 < ngrp)
            def _():
                fire_gathers(g + 1, obuf, osem)

            drain_gathers(g, buf, sem)
            run_scatters(g, buf)

        fire_gathers(0, rows.at[0], semA)

        @pl.loop(0, ngrp)
        def _grp(g):
            @pl.when(g % 2 == 0)
            def _():
                step(g, rows.at[0], semA, rows.at[1], semB)

            @pl.when(g % 2 == 1)
            def _():
                step(g, rows.at[1], semB, rows.at[0], semA)

        plsc.subcore_barrier()
        pltpu.sync_copy(acc.at[pl.ds(sid * T_ROWS, T_ROWS)], stage)
        pltpu.sync_copy(stage, out.at[cid, pl.ds(sid * T_ROWS, T_ROWS)])

    return pl.kernel(
        body,
        out_type=jax.ShapeDtypeStruct((NC, N, d), jnp.float32),
        mesh=_mesh(),
        compiler_params=_SC_PARAMS,
        scratch_types=[
            pltpu.VMEM((C0, CH), jnp.int32),
            pltpu.VMEM((C0, CH), jnp.int32),
            pltpu.VMEM((2, KF, CH, d), jnp.float32),
            pltpu.VMEM((T_ROWS, d), jnp.float32),
            pltpu.VMEM_SHARED((N, d), jnp.float32),
            pltpu.SemaphoreType.DMA,
            pltpu.SemaphoreType.DMA,
            pltpu.SemaphoreType.DMA,
        ],
    )


_sc_degree = _make_sc_degree()
_sc_scatter_h = _make_sc_scatter(HP)
_sc_scatter_z = _make_sc_scatter(ZP)


def _tc_matmul(x, W1p):
    def body(x_ref, w_ref, o_ref):
        o_ref[...] = jnp.dot(x_ref[...], w_ref[...], preferred_element_type=jnp.float32)

    return pl.pallas_call(
        body,
        grid=(_GRID,),
        in_specs=[
            pl.BlockSpec((_ROW_BLK, D_FEAT), lambda i: (i, 0)),
            pl.BlockSpec((D_FEAT, HP), lambda i: (0, 0)),
        ],
        out_specs=pl.BlockSpec((_ROW_BLK, HP), lambda i: (i, 0)),
        out_shape=jax.ShapeDtypeStruct((N, HP), jnp.float32),
    )(x, W1p)


def _tc_scale(deg2, u1):
    def body(dref, u, dinv_ref, v_ref):
        dd = dref[...]
        deg = 1.0 + dd[0][:, :1] + dd[1][:, :1]
        dinv = 1.0 / jnp.sqrt(deg)
        dinv_ref[...] = dinv
        v_ref[...] = u[...] * dinv

    return pl.pallas_call(
        body,
        grid=(_GRID,),
        in_specs=[
            pl.BlockSpec((NC, _ROW_BLK, DW), lambda i: (0, i, 0)),
            pl.BlockSpec((_ROW_BLK, HP), lambda i: (i, 0)),
        ],
        out_specs=[
            pl.BlockSpec((_ROW_BLK, 1), lambda i: (i, 0)),
            pl.BlockSpec((_ROW_BLK, HP), lambda i: (i, 0)),
        ],
        out_shape=[
            jax.ShapeDtypeStruct((N, 1), jnp.float32),
            jax.ShapeDtypeStruct((N, HP), jnp.float32),
        ],
    )(deg2, u1)


def _tc_mid(dinv, s1, v1, W2p, b1r):
    def body(di, s_ref, v_ref, w_ref, bias_ref, o_ref):
        dinv_blk = di[...]
        ss = s_ref[...]
        h = dinv_blk * (ss[0] + ss[1] + v_ref[...]) + bias_ref[...]
        u2 = jnp.dot(h, w_ref[...], preferred_element_type=jnp.float32)
        o_ref[...] = u2 * dinv_blk

    return pl.pallas_call(
        body,
        grid=(_GRID,),
        in_specs=[
            pl.BlockSpec((_ROW_BLK, 1), lambda i: (i, 0)),
            pl.BlockSpec((NC, _ROW_BLK, HP), lambda i: (0, i, 0)),
            pl.BlockSpec((_ROW_BLK, HP), lambda i: (i, 0)),
            pl.BlockSpec((HP, ZP), lambda i: (0, 0)),
            pl.BlockSpec((1, HP), lambda i: (0, 0)),
        ],
        out_specs=pl.BlockSpec((_ROW_BLK, ZP), lambda i: (i, 0)),
        out_shape=jax.ShapeDtypeStruct((N, ZP), jnp.float32),
    )(dinv, s1, v1, W2p, b1r)


def _tc_final(dinv, s2, v2, b2r):
    def body(di, s_ref, v_ref, bias_ref, o_ref):
        ss = s_ref[...]
        zfull = di[...] * (ss[0] + ss[1] + v_ref[...]) + bias_ref[...]
        o_ref[...] = zfull[:, :Z]

    return pl.pallas_call(
        body,
        grid=(_GRID,),
        in_specs=[
            pl.BlockSpec((_ROW_BLK, 1), lambda i: (i, 0)),
            pl.BlockSpec((NC, _ROW_BLK, ZP), lambda i: (0, i, 0)),
            pl.BlockSpec((_ROW_BLK, ZP), lambda i: (i, 0)),
            pl.BlockSpec((1, ZP), lambda i: (0, 0)),
        ],
        out_specs=pl.BlockSpec((_ROW_BLK, Z), lambda i: (i, 0)),
        out_shape=jax.ShapeDtypeStruct((N, Z), jnp.float32),
    )(dinv, s2, v2, b2r)


def kernel(x, edge_index, W1, b1, W2, b2):
    e2 = edge_index.astype(jnp.int32)
    e3 = e2.reshape(2, NCHUNK, CH)
    W1p = jnp.pad(W1, ((0, 0), (0, HP - H)))
    W2p = jnp.pad(W2, ((0, HP - H), (0, ZP - Z)))
    b1r = jnp.pad(b1, (0, HP - H)).reshape(1, HP)
    b2r = jnp.pad(b2, (0, ZP - Z)).reshape(1, ZP)
    ones_c = jnp.ones((_DCH, DW), jnp.float32)
    zdw = jnp.zeros((T_ROWS, DW), jnp.float32)
    zh = jnp.zeros((T_ROWS, HP), jnp.float32)
    zz = jnp.zeros((T_ROWS, ZP), jnp.float32)

    deg2 = _sc_degree(e2, ones_c, zdw)
    u1 = _tc_matmul(x, W1p)
    dinv, v1 = _tc_scale(deg2, u1)
    s1 = _sc_scatter_h(e3, v1, zh)
    v2 = _tc_mid(dinv, s1, v1, W2p, b1r)
    s2 = _sc_scatter_z(e3, v2, zz)
    return _tc_final(dinv, s2, v2, b2r)
